# baseline (speedup 1.0000x reference)
import functools

import jax
import jax.numpy as jnp
from jax import lax
from jax.experimental import pallas as pl
from jax.experimental.pallas import tpu as pltpu
from jax.experimental.pallas import tpu_sc as plsc

_DIM = 128
_NBATCH = 2
_V = _DIM * _DIM * _DIM
_N = _NBATCH * _V
_NS = 16
_PER_W = _V // _NS
_CH = 2048
_NG = _CH // 16
_GCH = _PER_W // _CH
_HALO = 16513
_PAD = 16544
_OFFS = (0, 1, 128, 129, 16384, 16385, 16512, 16513)
_W2 = _CH + 160

_mesh = plsc.VectorSubcoreMesh(
    core_axis_name="c", subcore_axis_name="s", num_cores=2, num_subcores=16
)
_params = pltpu.CompilerParams(
    needs_layout_passes=False, use_tc_tiling_on_sc=False)


@functools.partial(
    pl.kernel,
    out_type=(
        jax.ShapeDtypeStruct((_N, 8), jnp.float32),
        jax.ShapeDtypeStruct((_N,), jnp.float32),
    ),
    mesh=_mesh,
    scratch_types=[
        pltpu.VMEM((2, _W2), jnp.float32),
        pltpu.VMEM((2, _W2), jnp.float32),
        pltpu.VMEM((_CH, 8), jnp.float32),
        pltpu.VMEM((_CH, 8), jnp.float32),
        pltpu.SemaphoreType.DMA,
        pltpu.SemaphoreType.DMA,
        pltpu.SemaphoreType.DMA,
        pltpu.SemaphoreType.DMA,
        pltpu.VMEM((3 * _CH,), jnp.float32),
        pltpu.VMEM((3 * _CH,), jnp.float32),
        pltpu.VMEM((_CH,), jnp.int32),
        pltpu.VMEM((_CH,), jnp.int32),
        pltpu.VMEM((3, _CH), jnp.float32),
        pltpu.VMEM((3, _CH), jnp.float32),
        pltpu.VMEM((_CH, 8), jnp.float32),
        pltpu.VMEM((_CH, 8), jnp.float32),
        pltpu.VMEM((_CH,), jnp.float32),
        pltpu.SemaphoreType.DMA,
        pltpu.SemaphoreType.DMA,
        pltpu.SemaphoreType.DMA,
        pltpu.SemaphoreType.DMA,
    ],
    compiler_params=_params,
)
def _warp(ddf_hbm, img_hbm, tab_hbm, out_hbm,
          img_a, img_b, tab_a, tab_b, sem_ia, sem_ib, sem_ta, sem_tb,
          ddf_a, ddf_b, idx_a, idx_b, w_a, w_b, gat_a, gat_b,
          out_v, sem_da, sem_db, sem_ga, sem_gb):
    core = lax.axis_index("c")
    sub = lax.axis_index("s")
    tile_base = core * _V + sub * _PER_W
    iota = lax.iota(jnp.int32, 16)

    build_bufs = ((img_a, tab_a, sem_ia, sem_ta),
                  (img_b, tab_b, sem_ib, sem_tb))

    def img_win(g, w):
        base = tile_base + g * _CH + w * 16384
        return img_hbm.at[pl.ds(base, _W2)]

    def img_start(g, par):
        img_v, _, sem_i, _ = build_bufs[par]
        pltpu.async_copy(img_win(g, 0), img_v.at[0], sem_i)
        pltpu.async_copy(img_win(g, 1), img_v.at[1], sem_i)

    def img_wait(g, par):
        img_v, _, sem_i, _ = build_bufs[par]
        pltpu.make_async_copy(img_win(g, 0), img_v.at[0], sem_i).wait()
        pltpu.make_async_copy(img_win(g, 1), img_v.at[1], sem_i).wait()

    def tab_slice(g):
        return tab_hbm.at[pl.ds(tile_base + g * _CH, _CH), :]

    def build_half(g, par):
        img_v, tab_v, _, sem_t = build_bufs[par]
        img_wait(g, par)

        @pl.when(g >= 2)
        def _():
            pltpu.make_async_copy(tab_v, tab_slice(g - 2), sem_t).wait()

        @plsc.parallel_loop(0, _NG, unroll=8)
        def group_body(i):
            o = i * 16
            rows = o + iota
            for c, off in enumerate(_OFFS):
                w, off2 = (0, off) if off < 16384 else (1, off - 16384)
                v = img_v[w, pl.ds(o + off2, 16)]
                plsc.store_scatter(
                    tab_v, [rows, jnp.full((16,), c, jnp.int32)], v)

        plsc.subcore_barrier()
        pltpu.async_copy(tab_v, tab_slice(g), sem_t)
        img_start(jnp.minimum(g + 2, _GCH - 1), par)

    img_start(0, 0)
    img_start(1, 1)

    def build_pair(k, _):
        build_half(2 * k, 0)
        build_half(2 * k + 1, 1)
        return _

    lax.fori_loop(0, _GCH // 2, build_pair, None)
    pltpu.make_async_copy(tab_a, tab_slice(_GCH - 2), sem_ta).wait()
    pltpu.make_async_copy(tab_b, tab_slice(_GCH - 1), sem_tb).wait()
    img_wait(_GCH - 1, 0)
    img_wait(_GCH - 1, 1)

    plsc.subcore_barrier()

    def axis_split(coord_i, d, hi):
        loc = coord_i.astype(jnp.float32) + d
        loc = jnp.minimum(jnp.maximum(loc, 0.0), float(hi))
        f_i = jnp.minimum(loc.astype(jnp.int32), hi - 1)
        w = loc - f_i.astype(jnp.float32)
        return f_i, w

    batch_base = core << 21
    bufs = ((ddf_a, idx_a, w_a, gat_a, sem_da, sem_ga),
            (ddf_b, idx_b, w_b, gat_b, sem_db, sem_gb))

    def ddf_slice(g):
        return ddf_hbm.at[pl.ds((tile_base + g * _CH) * 3, 3 * _CH)]

    def ddf_start(g, par):
        pltpu.async_copy(ddf_slice(g), bufs[par][0], bufs[par][4])

    def ddf_wait(g, par):
        pltpu.make_async_copy(ddf_slice(g), bufs[par][0], bufs[par][4]).wait()

    def idx_phase(g, par):
        ddf_v, idx_v, w_v = bufs[par][0], bufs[par][1], bufs[par][2]
        base = tile_base + g * _CH

        def idx_group(o):
            sl = pl.ds(o, 16)
            dbase = (o >> 7) * 384 + (o & 127)
            dx = ddf_v[pl.ds(dbase, 16)]
            dy = ddf_v[pl.ds(dbase + 128, 16)]
            dz = ddf_v[pl.ds(dbase + 256, 16)]
            row = base + o
            ix, wx = axis_split((row >> 14) & 127, dx, 127)
            iy, wy = axis_split((row >> 7) & 127, dy, 127)
            iz, wz = axis_split((o & 127) + iota, dz, 127)
            idx_v[sl] = (batch_base + (ix << 14)) + ((iy << 7) + iz)
            w_v[0, sl] = wx
            w_v[1, sl] = wy
            w_v[2, sl] = wz

        @plsc.parallel_loop(0, _NG, unroll=8)
        def idx_body(i):
            idx_group(i * 16)

    def gather_start(par):
        pltpu.async_copy(
            tab_hbm.at[bufs[par][1]], bufs[par][3], bufs[par][5])

    def gather_wait(par):
        pltpu.make_async_copy(
            tab_hbm.at[bufs[par][1]], bufs[par][3], bufs[par][5]).wait()

    def blend_phase(g, par):
        w_v, gat_v = bufs[par][2], bufs[par][3]
        base = tile_base + g * _CH

        def blend_group(o):
            sl = pl.ds(o, 16)
            wx = w_v[0, sl]
            wy = w_v[1, sl]
            wz = w_v[2, sl]
            row = o + iota

            def corner(c):
                return plsc.load_gather(
                    gat_v, [row, jnp.full((16,), c, jnp.int32)])

            c00 = corner(0)
            c00 += wz * (corner(1) - c00)
            c01 = corner(2)
            c01 += wz * (corner(3) - c01)
            c10 = corner(4)
            c10 += wz * (corner(5) - c10)
            c11 = corner(6)
            c11 += wz * (corner(7) - c11)
            c0 = c00 + wy * (c01 - c00)
            c1 = c10 + wy * (c11 - c10)
            out_v[sl] = c0 + wx * (c1 - c0)

        @plsc.parallel_loop(0, _NG, unroll=8)
        def blend_body(i):
            blend_group(i * 16)

        plsc.subcore_barrier()
        pltpu.sync_copy(out_v, out_hbm.at[pl.ds(base, _CH)])

    ddf_start(0, 0)
    ddf_start(1, 1)

    def pipe_pair(k, _):
        g0 = 2 * k
        g1 = g0 + 1
        ddf_wait(g0, 0)
        idx_phase(g0, 0)
        plsc.subcore_barrier()
        gather_start(0)
        ddf_start(jnp.minimum(g0 + 2, _GCH - 1), 0)
        ddf_wait(g1, 1)
        idx_phase(g1, 1)
        plsc.subcore_barrier()
        gather_start(1)
        ddf_start(jnp.minimum(g1 + 2, _GCH - 1), 1)
        gather_wait(0)
        blend_phase(g0, 0)
        gather_wait(1)
        blend_phase(g1, 1)
        return _

    lax.fori_loop(0, _GCH // 2, pipe_pair, None)
    ddf_wait(_GCH - 1, 0)
    ddf_wait(_GCH - 1, 1)


def kernel(ddf, image):
    img_flat = image.reshape(-1)
    img_pad = jnp.concatenate(
        [img_flat, jnp.zeros((_PAD,), dtype=img_flat.dtype)])
    ddf_t = jnp.transpose(ddf, (0, 1, 2, 4, 3)).reshape(-1)
    _, out_flat = _warp(ddf_t, img_pad)
    return out_flat.reshape(image.shape)


# speedup vs baseline: 1.2716x; 1.2716x over previous
import functools

import jax
import jax.numpy as jnp
from jax import lax
from jax.experimental import pallas as pl
from jax.experimental.pallas import tpu as pltpu
from jax.experimental.pallas import tpu_sc as plsc

_DIM = 128
_NBATCH = 2
_V = _DIM * _DIM * _DIM
_N = _NBATCH * _V
_NS = 16
_PER_W = _V // _NS
_CH = 2048
_NG = _CH // 16
_GCH = _PER_W // _CH
_HALO = 16513
_PAD = 16544
_OFFS = (0, 1, 128, 129, 16384, 16385, 16512, 16513)
_W2 = _CH + 160
_TAIL0 = 18432

_mesh = plsc.VectorSubcoreMesh(
    core_axis_name="c", subcore_axis_name="s", num_cores=2, num_subcores=16
)
_params = pltpu.CompilerParams(
    needs_layout_passes=False, use_tc_tiling_on_sc=False)


@functools.partial(
    pl.kernel,
    out_type=(
        jax.ShapeDtypeStruct((_N, 8), jnp.float32),
        jax.ShapeDtypeStruct((_N,), jnp.float32),
    ),
    mesh=_mesh,
    scratch_types=[
        pltpu.VMEM((2, _W2), jnp.float32),
        pltpu.VMEM((2, _W2), jnp.float32),
        pltpu.VMEM((_CH, 8), jnp.float32),
        pltpu.VMEM((_CH, 8), jnp.float32),
        pltpu.SemaphoreType.DMA,
        pltpu.SemaphoreType.DMA,
        pltpu.SemaphoreType.DMA,
        pltpu.SemaphoreType.DMA,
        pltpu.VMEM((3 * _CH,), jnp.float32),
        pltpu.VMEM((3 * _CH,), jnp.float32),
        pltpu.VMEM((_CH,), jnp.int32),
        pltpu.VMEM((_CH,), jnp.int32),
        pltpu.VMEM((3, _CH), jnp.float32),
        pltpu.VMEM((3, _CH), jnp.float32),
        pltpu.VMEM((_CH, 8), jnp.float32),
        pltpu.VMEM((_CH, 8), jnp.float32),
        pltpu.VMEM((_CH,), jnp.float32),
        pltpu.SemaphoreType.DMA,
        pltpu.SemaphoreType.DMA,
        pltpu.SemaphoreType.DMA,
        pltpu.SemaphoreType.DMA,
    ],
    compiler_params=_params,
)
def _warp(ddf_hbm, img_hbm, tail_hbm, tab_hbm, out_hbm,
          img_a, img_b, tab_a, tab_b, sem_ia, sem_ib, sem_ta, sem_tb,
          ddf_a, ddf_b, idx_a, idx_b, w_a, w_b, gat_a, gat_b,
          out_v, sem_da, sem_db, sem_ga, sem_gb):
    core = lax.axis_index("c")
    sub = lax.axis_index("s")
    tile_base = core * _V + sub * _PER_W
    iota = lax.iota(jnp.int32, 16)

    build_bufs = ((img_a, tab_a, sem_ia, sem_ta),
                  (img_b, tab_b, sem_ib, sem_tb))

    def img_start(g, par):
        img_v, _, sem_i, _ = build_bufs[par]
        for w in (0, 1):
            s = tile_base + g * _CH + w * 16384

            @pl.when(s <= _N - _W2)
            def _():
                pltpu.async_copy(
                    img_hbm.at[pl.ds(s, _W2)], img_v.at[w], sem_i)

            @pl.when(s > _N - _W2)
            def _():
                pltpu.async_copy(
                    tail_hbm.at[pl.ds(s - (_N - _TAIL0), _W2)],
                    img_v.at[w], sem_i)

    def img_wait(par):
        img_v, _, sem_i, _ = build_bufs[par]
        for w in (0, 1):
            pltpu.make_async_copy(
                img_hbm.at[pl.ds(0, _W2)], img_v.at[w], sem_i).wait()

    def tab_slice(g):
        return tab_hbm.at[pl.ds(tile_base + g * _CH, _CH), :]

    def build_half(g, par):
        img_v, tab_v, _, sem_t = build_bufs[par]
        img_wait(par)

        @pl.when(g >= 2)
        def _():
            pltpu.make_async_copy(tab_v, tab_slice(g - 2), sem_t).wait()

        @plsc.parallel_loop(0, _NG, unroll=4)
        def group_body(i):
            o = i * 16
            rows = o + iota
            for c, off in enumerate(_OFFS):
                w, off2 = (0, off) if off < 16384 else (1, off - 16384)
                v = img_v[w, pl.ds(o + off2, 16)]
                plsc.store_scatter(
                    tab_v, [rows, jnp.full((16,), c, jnp.int32)], v)

        plsc.subcore_barrier()
        pltpu.async_copy(tab_v, tab_slice(g), sem_t)
        img_start(jnp.minimum(g + 2, _GCH - 1), par)

    img_start(0, 0)
    img_start(1, 1)

    def build_pair(k, _):
        build_half(2 * k, 0)
        build_half(2 * k + 1, 1)
        return _

    lax.fori_loop(0, _GCH // 2, build_pair, None)
    pltpu.make_async_copy(tab_a, tab_slice(_GCH - 2), sem_ta).wait()
    pltpu.make_async_copy(tab_b, tab_slice(_GCH - 1), sem_tb).wait()
    img_wait(0)
    img_wait(1)

    plsc.subcore_barrier()

    def axis_split(coord_i, d, hi):
        loc = coord_i.astype(jnp.float32) + d
        loc = jnp.minimum(jnp.maximum(loc, 0.0), float(hi))
        f_i = jnp.minimum(loc.astype(jnp.int32), hi - 1)
        w = loc - f_i.astype(jnp.float32)
        return f_i, w

    batch_base = core << 21
    bufs = ((ddf_a, idx_a, w_a, gat_a, sem_da, sem_ga),
            (ddf_b, idx_b, w_b, gat_b, sem_db, sem_gb))

    def ddf_slice(g):
        return ddf_hbm.at[pl.ds((tile_base + g * _CH) * 3, 3 * _CH)]

    def ddf_start(g, par):
        pltpu.async_copy(ddf_slice(g), bufs[par][0], bufs[par][4])

    def ddf_wait(g, par):
        pltpu.make_async_copy(ddf_slice(g), bufs[par][0], bufs[par][4]).wait()

    def idx_phase(g, par):
        ddf_v, idx_v, w_v = bufs[par][0], bufs[par][1], bufs[par][2]
        base = tile_base + g * _CH

        def idx_group(o):
            sl = pl.ds(o, 16)
            dbase = (o >> 7) * 384 + (o & 127)
            dx = ddf_v[pl.ds(dbase, 16)]
            dy = ddf_v[pl.ds(dbase + 128, 16)]
            dz = ddf_v[pl.ds(dbase + 256, 16)]
            row = base + o
            ix, wx = axis_split((row >> 14) & 127, dx, 127)
            iy, wy = axis_split((row >> 7) & 127, dy, 127)
            iz, wz = axis_split((o & 127) + iota, dz, 127)
            idx_v[sl] = (batch_base + (ix << 14)) + ((iy << 7) + iz)
            w_v[0, sl] = wx
            w_v[1, sl] = wy
            w_v[2, sl] = wz

        @plsc.parallel_loop(0, _NG, unroll=4)
        def idx_body(i):
            idx_group(i * 16)

    def gather_start(par):
        pltpu.async_copy(
            tab_hbm.at[bufs[par][1]], bufs[par][3], bufs[par][5])

    def gather_wait(par):
        pltpu.make_async_copy(
            tab_hbm.at[bufs[par][1]], bufs[par][3], bufs[par][5]).wait()

    def blend_phase(g, par):
        w_v, gat_v = bufs[par][2], bufs[par][3]
        base = tile_base + g * _CH

        def blend_group(o):
            sl = pl.ds(o, 16)
            wx = w_v[0, sl]
            wy = w_v[1, sl]
            wz = w_v[2, sl]
            row = o + iota

            def corner(c):
                return plsc.load_gather(
                    gat_v, [row, jnp.full((16,), c, jnp.int32)])

            c00 = corner(0)
            c00 += wz * (corner(1) - c00)
            c01 = corner(2)
            c01 += wz * (corner(3) - c01)
            c10 = corner(4)
            c10 += wz * (corner(5) - c10)
            c11 = corner(6)
            c11 += wz * (corner(7) - c11)
            c0 = c00 + wy * (c01 - c00)
            c1 = c10 + wy * (c11 - c10)
            out_v[sl] = c0 + wx * (c1 - c0)

        @plsc.parallel_loop(0, _NG, unroll=4)
        def blend_body(i):
            blend_group(i * 16)

        plsc.subcore_barrier()
        pltpu.sync_copy(out_v, out_hbm.at[pl.ds(base, _CH)])

    ddf_start(0, 0)
    ddf_start(1, 1)

    def pipe_pair(k, _):
        g0 = 2 * k
        g1 = g0 + 1
        ddf_wait(g0, 0)
        idx_phase(g0, 0)
        plsc.subcore_barrier()
        gather_start(0)
        ddf_start(jnp.minimum(g0 + 2, _GCH - 1), 0)
        ddf_wait(g1, 1)
        idx_phase(g1, 1)
        plsc.subcore_barrier()
        gather_start(1)
        ddf_start(jnp.minimum(g1 + 2, _GCH - 1), 1)
        gather_wait(0)
        blend_phase(g0, 0)
        gather_wait(1)
        blend_phase(g1, 1)
        return _

    lax.fori_loop(0, _GCH // 2, pipe_pair, None)
    ddf_wait(_GCH - 1, 0)
    ddf_wait(_GCH - 1, 1)


def kernel(ddf, image):
    img_flat = image.reshape(-1)
    tail = jnp.concatenate(
        [img_flat[_N - _TAIL0:], jnp.zeros((_PAD,), dtype=img_flat.dtype)])
    ddf_t = jnp.transpose(ddf, (0, 1, 2, 4, 3)).reshape(-1)
    _, out_flat = _warp(ddf_t, img_flat, tail)
    return out_flat.reshape(image.shape)


# speedup vs baseline: 1.3352x; 1.0500x over previous
import functools

import jax
import jax.numpy as jnp
from jax import lax
from jax.experimental import pallas as pl
from jax.experimental.pallas import tpu as pltpu
from jax.experimental.pallas import tpu_sc as plsc

_DIM = 128
_NBATCH = 2
_V = _DIM * _DIM * _DIM
_N = _NBATCH * _V
_NS = 16
_PER_W = _V // _NS
_CH = 2048
_NG = _CH // 16
_GCH = _PER_W // _CH
_HALO = 16513
_PAD = 16544
_OFFS = (0, 1, 128, 129, 16384, 16385, 16512, 16513)
_W2 = _CH + 160
_TAIL0 = 18432

_mesh = plsc.VectorSubcoreMesh(
    core_axis_name="c", subcore_axis_name="s", num_cores=2, num_subcores=16
)
_params = pltpu.CompilerParams(
    needs_layout_passes=False, use_tc_tiling_on_sc=False)


@functools.partial(
    pl.kernel,
    out_type=(
        jax.ShapeDtypeStruct((_N, 8), jnp.float32),
        jax.ShapeDtypeStruct((_N,), jnp.float32),
    ),
    mesh=_mesh,
    scratch_types=[
        pltpu.VMEM((2, _W2), jnp.float32),
        pltpu.VMEM((2, _W2), jnp.float32),
        pltpu.VMEM((_CH, 8), jnp.float32),
        pltpu.VMEM((_CH, 8), jnp.float32),
        pltpu.SemaphoreType.DMA,
        pltpu.SemaphoreType.DMA,
        pltpu.SemaphoreType.DMA,
        pltpu.SemaphoreType.DMA,
        pltpu.VMEM((3 * _CH,), jnp.float32),
        pltpu.VMEM((3 * _CH,), jnp.float32),
        pltpu.VMEM((_CH,), jnp.int32),
        pltpu.VMEM((_CH,), jnp.int32),
        pltpu.VMEM((3, _CH), jnp.float32),
        pltpu.VMEM((3, _CH), jnp.float32),
        pltpu.VMEM((_CH, 8), jnp.float32),
        pltpu.VMEM((_CH, 8), jnp.float32),
        pltpu.VMEM((_CH,), jnp.float32),
        pltpu.SemaphoreType.DMA,
        pltpu.SemaphoreType.DMA,
        pltpu.SemaphoreType.DMA,
        pltpu.SemaphoreType.DMA,
    ],
    compiler_params=_params,
)
def _warp(ddf_hbm, img_hbm, tail_hbm, tab_hbm, out_hbm,
          img_a, img_b, tab_a, tab_b, sem_ia, sem_ib, sem_ta, sem_tb,
          ddf_a, ddf_b, idx_a, idx_b, w_a, w_b, gat_a, gat_b,
          out_v, sem_da, sem_db, sem_ga, sem_gb):
    core = lax.axis_index("c")
    sub = lax.axis_index("s")
    tile_base = core * _V + sub * _PER_W
    iota = lax.iota(jnp.int32, 16)

    build_bufs = ((img_a, tab_a, sem_ia, sem_ta),
                  (img_b, tab_b, sem_ib, sem_tb))

    def img_start(g, par):
        img_v, _, sem_i, _ = build_bufs[par]
        for w in (0, 1):
            s = tile_base + g * _CH + w * 16384

            @pl.when(s <= _N - _W2)
            def _():
                pltpu.async_copy(
                    img_hbm.at[pl.ds(s, _W2)], img_v.at[w], sem_i)

            @pl.when(s > _N - _W2)
            def _():
                pltpu.async_copy(
                    tail_hbm.at[pl.ds(s - (_N - _TAIL0), _W2)],
                    img_v.at[w], sem_i)

    def img_wait(par):
        img_v, _, sem_i, _ = build_bufs[par]
        for w in (0, 1):
            pltpu.make_async_copy(
                img_hbm.at[pl.ds(0, _W2)], img_v.at[w], sem_i).wait()

    def tab_slice(g):
        return tab_hbm.at[pl.ds(tile_base + g * _CH, _CH), :]

    def build_half(g, par):
        img_v, tab_v, _, sem_t = build_bufs[par]
        img_wait(par)

        @pl.when(g >= 2)
        def _():
            pltpu.make_async_copy(tab_v, tab_slice(g - 2), sem_t).wait()

        @plsc.parallel_loop(0, _NG, unroll=2)
        def group_body(i):
            o = i * 16
            rows = o + iota
            for c, off in enumerate(_OFFS):
                w, off2 = (0, off) if off < 16384 else (1, off - 16384)
                v = img_v[w, pl.ds(o + off2, 16)]
                plsc.store_scatter(
                    tab_v, [rows, jnp.full((16,), c, jnp.int32)], v)

        plsc.subcore_barrier()
        pltpu.async_copy(tab_v, tab_slice(g), sem_t)
        img_start(jnp.minimum(g + 2, _GCH - 1), par)

    img_start(0, 0)
    img_start(1, 1)

    def build_pair(k, _):
        build_half(2 * k, 0)
        build_half(2 * k + 1, 1)
        return _

    lax.fori_loop(0, _GCH // 2, build_pair, None)
    pltpu.make_async_copy(tab_a, tab_slice(_GCH - 2), sem_ta).wait()
    pltpu.make_async_copy(tab_b, tab_slice(_GCH - 1), sem_tb).wait()
    img_wait(0)
    img_wait(1)

    plsc.subcore_barrier()

    def axis_split(coord_i, d, hi):
        loc = coord_i.astype(jnp.float32) + d
        loc = jnp.minimum(jnp.maximum(loc, 0.0), float(hi))
        f_i = jnp.minimum(loc.astype(jnp.int32), hi - 1)
        w = loc - f_i.astype(jnp.float32)
        return f_i, w

    batch_base = core << 21
    bufs = ((ddf_a, idx_a, w_a, gat_a, sem_da, sem_ga),
            (ddf_b, idx_b, w_b, gat_b, sem_db, sem_gb))

    def ddf_slice(g):
        return ddf_hbm.at[pl.ds((tile_base + g * _CH) * 3, 3 * _CH)]

    def ddf_start(g, par):
        pltpu.async_copy(ddf_slice(g), bufs[par][0], bufs[par][4])

    def ddf_wait(g, par):
        pltpu.make_async_copy(ddf_slice(g), bufs[par][0], bufs[par][4]).wait()

    def idx_phase(g, par):
        ddf_v, idx_v, w_v = bufs[par][0], bufs[par][1], bufs[par][2]
        base = tile_base + g * _CH

        def idx_group(o):
            sl = pl.ds(o, 16)
            dbase = (o >> 7) * 384 + (o & 127)
            dx = ddf_v[pl.ds(dbase, 16)]
            dy = ddf_v[pl.ds(dbase + 128, 16)]
            dz = ddf_v[pl.ds(dbase + 256, 16)]
            row = base + o
            ix, wx = axis_split((row >> 14) & 127, dx, 127)
            iy, wy = axis_split((row >> 7) & 127, dy, 127)
            iz, wz = axis_split((o & 127) + iota, dz, 127)
            idx_v[sl] = (batch_base + (ix << 14)) + ((iy << 7) + iz)
            w_v[0, sl] = wx
            w_v[1, sl] = wy
            w_v[2, sl] = wz

        @plsc.parallel_loop(0, _NG, unroll=2)
        def idx_body(i):
            idx_group(i * 16)

    def gather_start(par):
        pltpu.async_copy(
            tab_hbm.at[bufs[par][1]], bufs[par][3], bufs[par][5])

    def gather_wait(par):
        pltpu.make_async_copy(
            tab_hbm.at[bufs[par][1]], bufs[par][3], bufs[par][5]).wait()

    def blend_phase(g, par):
        w_v, gat_v = bufs[par][2], bufs[par][3]
        base = tile_base + g * _CH

        def blend_group(o):
            sl = pl.ds(o, 16)
            wx = w_v[0, sl]
            wy = w_v[1, sl]
            wz = w_v[2, sl]
            row = o + iota

            def corner(c):
                return plsc.load_gather(
                    gat_v, [row, jnp.full((16,), c, jnp.int32)])

            c00 = corner(0)
            c00 += wz * (corner(1) - c00)
            c01 = corner(2)
            c01 += wz * (corner(3) - c01)
            c10 = corner(4)
            c10 += wz * (corner(5) - c10)
            c11 = corner(6)
            c11 += wz * (corner(7) - c11)
            c0 = c00 + wy * (c01 - c00)
            c1 = c10 + wy * (c11 - c10)
            out_v[sl] = c0 + wx * (c1 - c0)

        @plsc.parallel_loop(0, _NG, unroll=2)
        def blend_body(i):
            blend_group(i * 16)

        plsc.subcore_barrier()
        pltpu.sync_copy(out_v, out_hbm.at[pl.ds(base, _CH)])

    ddf_start(0, 0)
    ddf_start(1, 1)

    def pipe_pair(k, _):
        g0 = 2 * k
        g1 = g0 + 1
        ddf_wait(g0, 0)
        idx_phase(g0, 0)
        plsc.subcore_barrier()
        gather_start(0)
        ddf_start(jnp.minimum(g0 + 2, _GCH - 1), 0)
        ddf_wait(g1, 1)
        idx_phase(g1, 1)
        plsc.subcore_barrier()
        gather_start(1)
        ddf_start(jnp.minimum(g1 + 2, _GCH - 1), 1)
        gather_wait(0)
        blend_phase(g0, 0)
        gather_wait(1)
        blend_phase(g1, 1)
        return _

    lax.fori_loop(0, _GCH // 2, pipe_pair, None)
    ddf_wait(_GCH - 1, 0)
    ddf_wait(_GCH - 1, 1)


def kernel(ddf, image):
    img_flat = image.reshape(-1)
    tail = jnp.concatenate(
        [img_flat[_N - _TAIL0:], jnp.zeros((_PAD,), dtype=img_flat.dtype)])
    ddf_t = jnp.transpose(ddf, (0, 1, 2, 4, 3)).reshape(-1)
    _, out_flat = _warp(ddf_t, img_flat, tail)
    return out_flat.reshape(image.shape)


# speedup vs baseline: 1.3416x; 1.0048x over previous
import functools

import jax
import jax.numpy as jnp
from jax import lax
from jax.experimental import pallas as pl
from jax.experimental.pallas import tpu as pltpu
from jax.experimental.pallas import tpu_sc as plsc

_DIM = 128
_NBATCH = 2
_V = _DIM * _DIM * _DIM
_N = _NBATCH * _V
_NS = 16
_PER_W = _V // _NS
_CH = 2048
_NG = _CH // 16
_GCH = _PER_W // _CH
_HALO = 16513
_PAD = 16544
_OFFS = (0, 1, 128, 129, 16384, 16385, 16512, 16513)
_W2 = _CH + 160
_TAIL0 = 18432

_mesh = plsc.VectorSubcoreMesh(
    core_axis_name="c", subcore_axis_name="s", num_cores=2, num_subcores=16
)
_params = pltpu.CompilerParams(
    needs_layout_passes=False, use_tc_tiling_on_sc=False)


@functools.partial(
    pl.kernel,
    out_type=(
        jax.ShapeDtypeStruct((_N, 8), jnp.float32),
        jax.ShapeDtypeStruct((_N,), jnp.float32),
    ),
    mesh=_mesh,
    scratch_types=[
        pltpu.VMEM((2, _W2), jnp.float32),
        pltpu.VMEM((2, _W2), jnp.float32),
        pltpu.VMEM((_CH, 8), jnp.float32),
        pltpu.VMEM((_CH, 8), jnp.float32),
        pltpu.SemaphoreType.DMA,
        pltpu.SemaphoreType.DMA,
        pltpu.SemaphoreType.DMA,
        pltpu.SemaphoreType.DMA,
        pltpu.VMEM((3 * _CH,), jnp.float32),
        pltpu.VMEM((3 * _CH,), jnp.float32),
        pltpu.VMEM((_CH,), jnp.int32),
        pltpu.VMEM((_CH,), jnp.int32),
        pltpu.VMEM((3, _CH), jnp.float32),
        pltpu.VMEM((3, _CH), jnp.float32),
        pltpu.VMEM((_CH, 8), jnp.float32),
        pltpu.VMEM((_CH, 8), jnp.float32),
        pltpu.VMEM((_CH,), jnp.float32),
        pltpu.SemaphoreType.DMA,
        pltpu.SemaphoreType.DMA,
        pltpu.SemaphoreType.DMA,
        pltpu.SemaphoreType.DMA,
    ],
    compiler_params=_params,
)
def _warp(ddf_hbm, img_hbm, tail_hbm, tab_hbm, out_hbm,
          img_a, img_b, tab_a, tab_b, sem_ia, sem_ib, sem_ta, sem_tb,
          ddf_a, ddf_b, idx_a, idx_b, w_a, w_b, gat_a, gat_b,
          out_v, sem_da, sem_db, sem_ga, sem_gb):
    core = lax.axis_index("c")
    sub = lax.axis_index("s")
    tile_base = core * _V + sub * _PER_W
    iota = lax.iota(jnp.int32, 16)

    build_bufs = ((img_a, tab_a, sem_ia, sem_ta),
                  (img_b, tab_b, sem_ib, sem_tb))

    def img_start(g, par):
        img_v, _, sem_i, _ = build_bufs[par]
        for w in (0, 1):
            s = tile_base + g * _CH + w * 16384

            @pl.when(s <= _N - _W2)
            def _():
                pltpu.async_copy(
                    img_hbm.at[pl.ds(s, _W2)], img_v.at[w], sem_i)

            @pl.when(s > _N - _W2)
            def _():
                pltpu.async_copy(
                    tail_hbm.at[pl.ds(s - (_N - _TAIL0), _W2)],
                    img_v.at[w], sem_i)

    def img_wait(par):
        img_v, _, sem_i, _ = build_bufs[par]
        for w in (0, 1):
            pltpu.make_async_copy(
                img_hbm.at[pl.ds(0, _W2)], img_v.at[w], sem_i).wait()

    def tab_slice(g):
        return tab_hbm.at[pl.ds(tile_base + g * _CH, _CH), :]

    def build_half(g, par):
        img_v, tab_v, _, sem_t = build_bufs[par]
        img_wait(par)

        @pl.when(g >= 2)
        def _():
            pltpu.make_async_copy(tab_v, tab_slice(g - 2), sem_t).wait()

        @plsc.parallel_loop(0, _NG, unroll=1)
        def group_body(i):
            o = i * 16
            rows = o + iota
            for c, off in enumerate(_OFFS):
                w, off2 = (0, off) if off < 16384 else (1, off - 16384)
                v = img_v[w, pl.ds(o + off2, 16)]
                plsc.store_scatter(
                    tab_v, [rows, jnp.full((16,), c, jnp.int32)], v)

        plsc.subcore_barrier()
        pltpu.async_copy(tab_v, tab_slice(g), sem_t)
        img_start(jnp.minimum(g + 2, _GCH - 1), par)

    img_start(0, 0)
    img_start(1, 1)

    def build_pair(k, _):
        build_half(2 * k, 0)
        build_half(2 * k + 1, 1)
        return _

    lax.fori_loop(0, _GCH // 2, build_pair, None)
    pltpu.make_async_copy(tab_a, tab_slice(_GCH - 2), sem_ta).wait()
    pltpu.make_async_copy(tab_b, tab_slice(_GCH - 1), sem_tb).wait()
    img_wait(0)
    img_wait(1)

    plsc.subcore_barrier()

    def axis_split(coord_i, d, hi):
        loc = coord_i.astype(jnp.float32) + d
        loc = jnp.minimum(jnp.maximum(loc, 0.0), float(hi))
        f_i = jnp.minimum(loc.astype(jnp.int32), hi - 1)
        w = loc - f_i.astype(jnp.float32)
        return f_i, w

    batch_base = core << 21
    bufs = ((ddf_a, idx_a, w_a, gat_a, sem_da, sem_ga),
            (ddf_b, idx_b, w_b, gat_b, sem_db, sem_gb))

    def ddf_slice(g):
        return ddf_hbm.at[pl.ds((tile_base + g * _CH) * 3, 3 * _CH)]

    def ddf_start(g, par):
        pltpu.async_copy(ddf_slice(g), bufs[par][0], bufs[par][4])

    def ddf_wait(g, par):
        pltpu.make_async_copy(ddf_slice(g), bufs[par][0], bufs[par][4]).wait()

    def idx_phase(g, par):
        ddf_v, idx_v, w_v = bufs[par][0], bufs[par][1], bufs[par][2]
        base = tile_base + g * _CH

        def idx_group(o):
            sl = pl.ds(o, 16)
            dbase = (o >> 7) * 384 + (o & 127)
            dx = ddf_v[pl.ds(dbase, 16)]
            dy = ddf_v[pl.ds(dbase + 128, 16)]
            dz = ddf_v[pl.ds(dbase + 256, 16)]
            row = base + o
            ix, wx = axis_split((row >> 14) & 127, dx, 127)
            iy, wy = axis_split((row >> 7) & 127, dy, 127)
            iz, wz = axis_split((o & 127) + iota, dz, 127)
            idx_v[sl] = (batch_base + (ix << 14)) + ((iy << 7) + iz)
            w_v[0, sl] = wx
            w_v[1, sl] = wy
            w_v[2, sl] = wz

        @plsc.parallel_loop(0, _NG, unroll=1)
        def idx_body(i):
            idx_group(i * 16)

    def gather_start(par):
        pltpu.async_copy(
            tab_hbm.at[bufs[par][1]], bufs[par][3], bufs[par][5])

    def gather_wait(par):
        pltpu.make_async_copy(
            tab_hbm.at[bufs[par][1]], bufs[par][3], bufs[par][5]).wait()

    def blend_phase(g, par):
        w_v, gat_v = bufs[par][2], bufs[par][3]
        base = tile_base + g * _CH

        def blend_group(o):
            sl = pl.ds(o, 16)
            wx = w_v[0, sl]
            wy = w_v[1, sl]
            wz = w_v[2, sl]
            row = o + iota

            def corner(c):
                return plsc.load_gather(
                    gat_v, [row, jnp.full((16,), c, jnp.int32)])

            c00 = corner(0)
            c00 += wz * (corner(1) - c00)
            c01 = corner(2)
            c01 += wz * (corner(3) - c01)
            c10 = corner(4)
            c10 += wz * (corner(5) - c10)
            c11 = corner(6)
            c11 += wz * (corner(7) - c11)
            c0 = c00 + wy * (c01 - c00)
            c1 = c10 + wy * (c11 - c10)
            out_v[sl] = c0 + wx * (c1 - c0)

        @plsc.parallel_loop(0, _NG, unroll=1)
        def blend_body(i):
            blend_group(i * 16)

        plsc.subcore_barrier()
        pltpu.sync_copy(out_v, out_hbm.at[pl.ds(base, _CH)])

    ddf_start(0, 0)
    ddf_start(1, 1)

    def pipe_pair(k, _):
        g0 = 2 * k
        g1 = g0 + 1
        ddf_wait(g0, 0)
        idx_phase(g0, 0)
        plsc.subcore_barrier()
        gather_start(0)
        ddf_start(jnp.minimum(g0 + 2, _GCH - 1), 0)
        ddf_wait(g1, 1)
        idx_phase(g1, 1)
        plsc.subcore_barrier()
        gather_start(1)
        ddf_start(jnp.minimum(g1 + 2, _GCH - 1), 1)
        gather_wait(0)
        blend_phase(g0, 0)
        gather_wait(1)
        blend_phase(g1, 1)
        return _

    lax.fori_loop(0, _GCH // 2, pipe_pair, None)
    ddf_wait(_GCH - 1, 0)
    ddf_wait(_GCH - 1, 1)


def kernel(ddf, image):
    img_flat = image.reshape(-1)
    tail = jnp.concatenate(
        [img_flat[_N - _TAIL0:], jnp.zeros((_PAD,), dtype=img_flat.dtype)])
    ddf_t = jnp.transpose(ddf, (0, 1, 2, 4, 3)).reshape(-1)
    _, out_flat = _warp(ddf_t, img_flat, tail)
    return out_flat.reshape(image.shape)
